# baseline (speedup 1.0000x reference)
import jax
import jax.numpy as jnp
from jax import lax
from jax.experimental import pallas as pl
from jax.experimental.pallas import tpu as pltpu
from jax.experimental.pallas import tpu_sc as plsc

C = 1000
D = 1000
B = 16384
W = 512
PAD = 16
WP = W + PAD
CHUNK = 32
NCHUNK = 32
CB = 64
COL_OVL = D - W


def _body(images, labels2d, protos, alphas, betas, out, acc, img_buf, lab,
          pbuf, abuf, sbuf, bbuf):
    c = lax.axis_index("c")
    s = lax.axis_index("s")
    rc = c * COL_OVL

    zeros16 = jnp.zeros((16,), jnp.float32)

    def zrow(r, carry):
        for g in range(WP // 16):
            img_buf[r, pl.ds(g * 16, 16)] = zeros16
        return carry

    lax.fori_loop(0, CHUNK, zrow, 0)
    c0z = jnp.minimum(s * CB, C - CB)
    pltpu.sync_copy(img_buf.at[pl.ds(0, CB), :], acc.at[pl.ds(c0z, CB), :])
    plsc.subcore_barrier()

    onehot = jnp.where(lax.iota(jnp.int32, 16) == 0, 1.0, 0.0).astype(
        jnp.float32)

    def crow(r, carry):
        img_buf[r, pl.ds(0, 16)] = onehot
        return carry

    lax.fori_loop(0, CHUNK, crow, 0)

    pltpu.sync_copy(labels2d.at[pl.ds(s * NCHUNK * CHUNK, NCHUNK * CHUNK), :],
                    lab)

    def chunk_body(j, carry):
        r0 = s * (NCHUNK * CHUNK) + j * CHUNK
        pltpu.sync_copy(images.at[pl.ds(r0, CHUNK), pl.ds(rc, W)],
                        img_buf.at[:, pl.ds(PAD, W)])
        for r in range(CHUNK):
            pltpu.sync_copy(img_buf.at[pl.ds(r, 1), :],
                            acc.at[lab.at[j * CHUNK + r]], add=True)
        return carry

    lax.fori_loop(0, NCHUNK, chunk_body, 0)
    plsc.subcore_barrier()

    c0 = jnp.minimum(s * CB, C - CB)
    pltpu.sync_copy(protos.at[pl.ds(c0, CB), pl.ds(rc, W)], pbuf)
    pltpu.sync_copy(acc.at[pl.ds(c0, CB), :], abuf)
    pltpu.sync_copy(alphas.at[pl.ds(c0, CB), :], sbuf)
    pltpu.sync_copy(betas.at[pl.ds(c0, CB), :], bbuf)

    zero16i = jnp.zeros((16,), jnp.int32)

    def cls_body(i, carry):
        ii = jnp.full((16,), i, jnp.int32)
        av = plsc.load_gather(sbuf, [ii, zero16i])
        bv = plsc.load_gather(bbuf, [ii, zero16i])
        cnt = plsc.load_gather(abuf, [ii, zero16i])
        pres = cnt > 0.0
        for g in range(W // 16):
            p = pbuf[i, pl.ds(g * 16, 16)]
            a = abuf[i, pl.ds(PAD + g * 16, 16)]
            u = av * p + bv * a
            pbuf[i, pl.ds(g * 16, 16)] = jnp.where(pres, u, p)
        return carry

    lax.fori_loop(0, CB, cls_body, 0)
    pltpu.sync_copy(pbuf, out.at[pl.ds(c0, CB), pl.ds(c * COL_OVL, W)])


def kernel(images, labels, prototypes, cur_class_size):
    labels2d = labels.astype(jnp.int32).reshape(B, 1)
    denom = cur_class_size + float(B)
    alphas = cur_class_size / denom
    betas = 1.0 / denom
    mesh = plsc.VectorSubcoreMesh(core_axis_name="c", subcore_axis_name="s")
    f = pl.kernel(
        _body,
        out_type=jax.ShapeDtypeStruct((C, D), jnp.float32),
        mesh=mesh,
        compiler_params=pltpu.CompilerParams(use_tc_tiling_on_sc=False, needs_layout_passes=False),
        scratch_types=[
            pltpu.VMEM_SHARED((C, WP), jnp.float32),
            pltpu.VMEM((CHUNK, WP), jnp.float32),
            pltpu.VMEM((NCHUNK * CHUNK, 1), jnp.int32),
            pltpu.VMEM((CB, W), jnp.float32),
            pltpu.VMEM((CB, WP), jnp.float32),
            pltpu.VMEM((CB, 1), jnp.float32),
            pltpu.VMEM((CB, 1), jnp.float32),
        ],
    )
    return f(images, labels2d, prototypes, alphas, betas)


# speedup vs baseline: 1.2435x; 1.2435x over previous
import jax
import jax.numpy as jnp
from jax import lax
from jax.experimental import pallas as pl
from jax.experimental.pallas import tpu as pltpu
from jax.experimental.pallas import tpu_sc as plsc

C = 1000
D = 1000
B = 16384
W = 512
PAD = 16
WP = W + PAD
CHUNK = 16
NCHUNK = 64
CB = 64
HB = 32
COL_OVL = D - W


def _body(images, labels2d, protos, alphas, betas, out, acc, img_buf, lab,
          pbuf, abuf, sbuf, bbuf, img_sem, sc_sem):
    c = lax.axis_index("c")
    s = lax.axis_index("s")
    rc = c * COL_OVL

    zeros16 = jnp.zeros((16,), jnp.float32)

    def zrow(r, carry):
        for g in range(WP // 16):
            img_buf[r, pl.ds(g * 16, 16)] = zeros16
        return carry

    lax.fori_loop(0, 2 * CHUNK, zrow, 0)
    c0z = jnp.minimum(s * CB, C - CB)
    pltpu.sync_copy(img_buf.at[pl.ds(0, HB), :], acc.at[pl.ds(c0z, HB), :])
    pltpu.sync_copy(img_buf.at[pl.ds(0, HB), :],
                    acc.at[pl.ds(c0z + HB, HB), :])
    plsc.subcore_barrier()

    onehot = jnp.where(lax.iota(jnp.int32, 16) == 0, 1.0, 0.0).astype(
        jnp.float32)

    def crow(r, carry):
        img_buf[r, pl.ds(0, 16)] = onehot
        return carry

    lax.fori_loop(0, 2 * CHUNK, crow, 0)

    base = s * (NCHUNK * CHUNK)
    pltpu.sync_copy(labels2d.at[pl.ds(base, NCHUNK * CHUNK), :], lab)

    pltpu.sync_copy(images.at[pl.ds(base, CHUNK), pl.ds(rc, W)],
                    img_buf.at[pl.ds(0, CHUNK), pl.ds(PAD, W)])

    def chunk_body(j, carry):
        b = lax.rem(j, 2) * CHUNK
        nb = lax.rem(j + 1, 2) * CHUNK
        hs = []
        for r in range(CHUNK):
            hs.append(pltpu.async_copy(
                img_buf.at[pl.ds(b + r, 1), :],
                acc.at[lab.at[j * CHUNK + r]], sc_sem, add=True))
        jn = jnp.minimum(j + 1, NCHUNK - 1)
        hi = pltpu.async_copy(
            images.at[pl.ds(base + jn * CHUNK, CHUNK), pl.ds(rc, W)],
            img_buf.at[pl.ds(nb, CHUNK), pl.ds(PAD, W)], img_sem)
        for h in hs:
            h.wait()
        hi.wait()
        return carry

    lax.fori_loop(0, NCHUNK, chunk_body, 0)
    plsc.subcore_barrier()

    c0 = jnp.minimum(s * CB, C - CB)
    zero16i = jnp.zeros((16,), jnp.int32)
    for h in range(CB // HB):
        ch = c0 + h * HB
        pltpu.sync_copy(protos.at[pl.ds(ch, HB), pl.ds(rc, W)], pbuf)
        pltpu.sync_copy(acc.at[pl.ds(ch, HB), :], abuf)
        pltpu.sync_copy(alphas.at[pl.ds(ch, HB), :], sbuf)
        pltpu.sync_copy(betas.at[pl.ds(ch, HB), :], bbuf)

        def cls_body(i, carry):
            ii = jnp.full((16,), i, jnp.int32)
            av = plsc.load_gather(sbuf, [ii, zero16i])
            bv = plsc.load_gather(bbuf, [ii, zero16i])
            cnt = plsc.load_gather(abuf, [ii, zero16i])
            pres = cnt > 0.0
            for g in range(W // 16):
                p = pbuf[i, pl.ds(g * 16, 16)]
                a = abuf[i, pl.ds(PAD + g * 16, 16)]
                u = av * p + bv * a
                pbuf[i, pl.ds(g * 16, 16)] = jnp.where(pres, u, p)
            return carry

        lax.fori_loop(0, HB, cls_body, 0)
        pltpu.sync_copy(pbuf, out.at[pl.ds(ch, HB), pl.ds(c * COL_OVL, W)])


def kernel(images, labels, prototypes, cur_class_size):
    labels2d = labels.astype(jnp.int32).reshape(B, 1)
    denom = cur_class_size + float(B)
    alphas = cur_class_size / denom
    betas = 1.0 / denom
    mesh = plsc.VectorSubcoreMesh(core_axis_name="c", subcore_axis_name="s")
    f = pl.kernel(
        _body,
        out_type=jax.ShapeDtypeStruct((C, D), jnp.float32),
        mesh=mesh,
        compiler_params=pltpu.CompilerParams(
            use_tc_tiling_on_sc=False, needs_layout_passes=False),
        scratch_types=[
            pltpu.VMEM_SHARED((C, WP), jnp.float32),
            pltpu.VMEM((2 * CHUNK, WP), jnp.float32),
            pltpu.VMEM((NCHUNK * CHUNK, 1), jnp.int32),
            pltpu.VMEM((HB, W), jnp.float32),
            pltpu.VMEM((HB, WP), jnp.float32),
            pltpu.VMEM((HB, 1), jnp.float32),
            pltpu.VMEM((HB, 1), jnp.float32),
            pltpu.SemaphoreType.DMA,
            pltpu.SemaphoreType.DMA,
        ],
    )
    return f(images, labels2d, prototypes, alphas, betas)


# speedup vs baseline: 1.2929x; 1.0397x over previous
import jax
import jax.numpy as jnp
from jax import lax
from jax.experimental import pallas as pl
from jax.experimental.pallas import tpu as pltpu
from jax.experimental.pallas import tpu_sc as plsc

C = 1000
D = 1000
B = 16384
W = 512
PAD = 16
WP = W + PAD
TRASH = C
CHUNK = 16
NCHUNK = 64
CB = 64
HB = 32
SPARE = 16
COL_OVL = D - W


def _body(images, labels, protos, alphas, betas, out, acc, img_buf, labs,
          lab1d, tmp1, pbuf, abuf, sbuf, bbuf, img_sem, sc_sem):
    c = lax.axis_index("c")
    s = lax.axis_index("s")
    rc = c * COL_OVL

    zeros16 = jnp.zeros((16,), jnp.float32)

    def zrow(r, carry):
        for g in range(WP // 16):
            img_buf[r, pl.ds(g * 16, 16)] = zeros16
        return carry

    lax.fori_loop(0, 2 * CHUNK, zrow, 0)
    c0z = jnp.minimum(s * CB, C - CB)
    pltpu.sync_copy(img_buf.at[pl.ds(0, HB), :], acc.at[pl.ds(c0z, HB), :])
    pltpu.sync_copy(img_buf.at[pl.ds(0, HB), :],
                    acc.at[pl.ds(c0z + HB, HB), :])
    plsc.subcore_barrier()

    onehot = jnp.where(lax.iota(jnp.int32, 16) == 0, 1.0, 0.0).astype(
        jnp.float32)

    def crow(r, carry):
        img_buf[r, pl.ds(0, 16)] = onehot
        return carry

    lax.fori_loop(0, 2 * CHUNK, crow, 0)

    base = s * (NCHUNK * CHUNK)
    pltpu.sync_copy(labels.at[pl.ds(base, NCHUNK * CHUNK)], lab1d)

    pltpu.sync_copy(images.at[pl.ds(base, CHUNK), pl.ds(rc, W)],
                    img_buf.at[pl.ds(0, CHUNK), pl.ds(PAD, W)])

    iota = lax.iota(jnp.int32, 16)

    def chunk_body(j, carry):
        b = lax.rem(j, 2) * CHUNK
        nb = lax.rem(j + 1, 2) * CHUNK

        lv = lab1d[pl.ds(j * CHUNK, 16)]
        srt_k, srt_v = plsc.sort_key_val(lv, iota)
        tmp1[pl.ds(0, 16)] = srt_k
        prev = plsc.load_gather(tmp1, [jnp.maximum(iota - 1, 0)])
        newgrp = jnp.logical_or(srt_k != prev, iota == 0)
        grp_start = plsc.cummax(jnp.where(newgrp, iota, 0))
        rank_sorted = iota - grp_start
        plsc.store_scatter(tmp1, [srt_v], rank_sorted)
        ranks = tmp1[pl.ds(0, 16)]
        maxrank = jnp.max(ranks)

        ring = lax.rem(j, 8)
        labs[ring, pl.ds(0, 16)] = jnp.where(ranks == 0, lv, TRASH)
        h1 = pltpu.async_copy(img_buf.at[pl.ds(b, CHUNK), :],
                              acc.at[labs.at[ring]], sc_sem, add=True)
        jn = jnp.minimum(j + 1, NCHUNK - 1)
        h2 = pltpu.async_copy(
            images.at[pl.ds(base + jn * CHUNK, CHUNK), pl.ds(rc, W)],
            img_buf.at[pl.ds(nb, CHUNK), pl.ds(PAD, W)], img_sem)

        @pl.when(maxrank > 0)
        def _dups():
            def round_body(k, carry2):
                labs[SPARE, pl.ds(0, 16)] = jnp.where(ranks == k, lv, TRASH)
                pltpu.sync_copy(img_buf.at[pl.ds(b, CHUNK), :],
                                acc.at[labs.at[SPARE]], add=True)
                return carry2

            lax.fori_loop(1, maxrank + 1, round_body, 0)

        h1.wait()
        h2.wait()
        return carry

    lax.fori_loop(0, NCHUNK, chunk_body, 0)
    plsc.subcore_barrier()

    c0 = jnp.minimum(s * CB, C - CB)
    zero16i = jnp.zeros((16,), jnp.int32)
    for h in range(CB // HB):
        ch = c0 + h * HB
        pltpu.sync_copy(protos.at[pl.ds(ch, HB), pl.ds(rc, W)], pbuf)
        pltpu.sync_copy(acc.at[pl.ds(ch, HB), :], abuf)
        pltpu.sync_copy(alphas.at[pl.ds(ch, HB), :], sbuf)
        pltpu.sync_copy(betas.at[pl.ds(ch, HB), :], bbuf)

        def cls_body(i, carry):
            ii = jnp.full((16,), i, jnp.int32)
            av = plsc.load_gather(sbuf, [ii, zero16i])
            bv = plsc.load_gather(bbuf, [ii, zero16i])
            cnt = plsc.load_gather(abuf, [ii, zero16i])
            pres = cnt > 0.0
            for g in range(W // 16):
                p = pbuf[i, pl.ds(g * 16, 16)]
                a = abuf[i, pl.ds(PAD + g * 16, 16)]
                u = av * p + bv * a
                pbuf[i, pl.ds(g * 16, 16)] = jnp.where(pres, u, p)
            return carry

        lax.fori_loop(0, HB, cls_body, 0)
        pltpu.sync_copy(pbuf, out.at[pl.ds(ch, HB), pl.ds(c * COL_OVL, W)])


def kernel(images, labels, prototypes, cur_class_size):
    labels = labels.astype(jnp.int32)
    denom = cur_class_size + float(B)
    alphas = cur_class_size / denom
    betas = 1.0 / denom
    mesh = plsc.VectorSubcoreMesh(core_axis_name="c", subcore_axis_name="s")
    f = pl.kernel(
        _body,
        out_type=jax.ShapeDtypeStruct((C, D), jnp.float32),
        mesh=mesh,
        compiler_params=pltpu.CompilerParams(
            use_tc_tiling_on_sc=False, needs_layout_passes=False),
        scratch_types=[
            pltpu.VMEM_SHARED((C + 8, WP), jnp.float32),
            pltpu.VMEM((2 * CHUNK, WP), jnp.float32),
            pltpu.VMEM((SPARE + 8, 16), jnp.int32),
            pltpu.VMEM((NCHUNK * CHUNK,), jnp.int32),
            pltpu.VMEM((16,), jnp.int32),
            pltpu.VMEM((HB, W), jnp.float32),
            pltpu.VMEM((HB, WP), jnp.float32),
            pltpu.VMEM((HB, 1), jnp.float32),
            pltpu.VMEM((HB, 1), jnp.float32),
            pltpu.SemaphoreType.DMA,
            pltpu.SemaphoreType.DMA,
        ],
    )
    return f(images, labels, prototypes, alphas, betas)
